# baseline (speedup 1.0000x reference)
import functools

import jax
import jax.numpy as jnp
from jax.experimental import pallas as pl
from jax.experimental.pallas import tpu as pltpu


def _moe_body(x_ref, aff_ref, idx_ref, gup_ref, down_ref, out_ref,
              w_ref, gacc_ref, uacc_ref, *, top_k, num_e, ns, tile_i,
              inter_dim):
    e = pl.program_id(0)
    s = pl.program_id(1)

    @pl.when((e == 0) & (s == 0))
    def _init():
        t, ne = w_ref.shape
        idx = idx_ref[...]
        erange = jax.lax.broadcasted_iota(jnp.int32, (t, ne), 1)
        mask = jnp.zeros((t, ne), jnp.float32)
        for k in range(top_k):
            mask = mask + (idx[:, k][:, None] == erange).astype(jnp.float32)
        w = jnp.where(mask == 0.0, 0.0, aff_ref[...])
        denom = jnp.maximum(jnp.sum(jnp.abs(w), axis=1, keepdims=True), 1e-12)
        w_ref[...] = w / denom
        out_ref[...] = jnp.zeros_like(out_ref)

    @pl.when(e < num_e)
    def _phase1():
        par = jax.lax.rem(e, 2)
        xb = x_ref[...].astype(jnp.bfloat16)
        res = jnp.dot(xb, gup_ref[0].astype(jnp.bfloat16),
                      preferred_element_type=jnp.float32)
        g = res[:, :inter_dim]
        u = res[:, inter_dim:]

        @pl.when(s == 0)
        def _():
            gacc_ref[par] = g
            uacc_ref[par] = u

        @pl.when(s != 0)
        def _():
            gacc_ref[par] += g
            uacc_ref[par] += u

    for i in range(ns):
        @pl.when((e >= 1) & (s == i))
        def _phase2(i=i):
            par2 = jax.lax.rem(e + 1, 2)
            gs = gacc_ref[par2, :, i * tile_i:(i + 1) * tile_i]
            us = uacc_ref[par2, :, i * tile_i:(i + 1) * tile_i]
            inter = (gs * jax.lax.logistic(gs) * us).astype(jnp.bfloat16)
            part = jnp.dot(inter, down_ref[0].astype(jnp.bfloat16),
                           preferred_element_type=jnp.float32)
            w_full = w_ref[...]
            col = jax.lax.broadcasted_iota(jnp.int32, w_full.shape, 1)
            we = jnp.sum(jnp.where(col == e - 1, w_full, 0.0), axis=1,
                         keepdims=True)
            out_ref[...] += part * we


def kernel(hidden_states, expert_affinities, expert_index, gate_up_proj,
           down_proj):
    t, h = hidden_states.shape
    num_e = expert_affinities.shape[1]
    top_k = expert_index.shape[1]
    inter_dim = down_proj.shape[1]
    ns = 8
    tile_h = h // ns
    tile_i = inter_dim // ns
    expert_index = expert_index.astype(jnp.int32)

    def gup_map(e, s):
        ee = jnp.minimum(e, num_e - 1)
        ss = jnp.where(e >= num_e, ns - 1, s)
        return (ee, ss, 0)

    def x_map(e, s):
        return (0, jnp.where(e >= num_e, ns - 1, s))

    def down_map(e, s):
        return (jnp.maximum(e - 1, 0), jnp.where(e == 0, 0, s), 0)

    body = functools.partial(_moe_body, top_k=top_k, num_e=num_e, ns=ns,
                             tile_i=tile_i, inter_dim=inter_dim)
    return pl.pallas_call(
        body,
        grid=(num_e + 1, ns),
        in_specs=[
            pl.BlockSpec((t, tile_h), x_map),
            pl.BlockSpec((t, num_e), lambda e, s: (0, 0)),
            pl.BlockSpec((t, top_k), lambda e, s: (0, 0)),
            pl.BlockSpec((1, tile_h, 2 * inter_dim), gup_map),
            pl.BlockSpec((1, tile_i, h), down_map),
        ],
        out_specs=pl.BlockSpec((t, h), lambda e, s: (0, 0)),
        out_shape=jax.ShapeDtypeStruct((t, h), jnp.float32),
        scratch_shapes=[
            pltpu.VMEM((t, num_e), jnp.float32),
            pltpu.VMEM((2, t, inter_dim), jnp.float32),
            pltpu.VMEM((2, t, inter_dim), jnp.float32),
        ],
    )(hidden_states, expert_affinities, expert_index, gate_up_proj,
      down_proj)
